# baseline (speedup 1.0000x reference)
import functools

import jax
import jax.numpy as jnp
from jax import lax
from jax.experimental import pallas as pl
from jax.experimental.pallas import tpu as pltpu, tpu_sc as plsc

BATCH = 16384
DIM = 32
NW = 32
B_PER_W = BATCH // NW
CHUNK = 128
NCH = B_PER_W // CHUNK
LINE = 128


def _body(uline_hbm, uoff_hbm, sline_hbm, soff_hbm, utab_hbm, itab_hbm,
          out_hbm, uline_v, uoff_v, sline_v, soff_v, ubuf, vbuf, out_v,
          gsem):
    wid = lax.axis_index("s") * 2 + lax.axis_index("c")
    base = wid * NCH

    pltpu.sync_copy(uline_hbm.at[pl.ds(base, NCH)], uline_v)
    pltpu.sync_copy(sline_hbm.at[pl.ds(base, NCH)], sline_v)
    pltpu.sync_copy(uoff_hbm.at[pl.ds(base, NCH)], uoff_v)
    pltpu.sync_copy(soff_hbm.at[pl.ds(base, NCH)], soff_v)

    def start(j):
        slot = j % 2
        cu = pltpu.async_copy(utab_hbm.at[uline_v.at[j]], ubuf.at[slot], gsem)
        cv = pltpu.async_copy(itab_hbm.at[sline_v.at[j]], vbuf.at[slot], gsem)
        return cu, cv

    lanes = lax.iota(jnp.int32, 16)

    def compute(j):
        slot = j % 2
        for g in range(CHUNK // 16):
            res = jnp.zeros((16,), jnp.float32)
            uoffs = uoff_v[j, pl.ds(g * 16, 16)]
            soffs = soff_v[j, pl.ds(g * 16, 16)]
            for i in range(16):
                b = g * 16 + i
                uo = uoffs[i]
                so = soffs[i]
                u0 = ubuf[slot, b, pl.ds(uo, 16)]
                u1 = ubuf[slot, b, pl.ds(uo + 16, 16)]
                v0 = vbuf[slot, b, pl.ds(so, 16)]
                v1 = vbuf[slot, b, pl.ds(so + 16, 16)]
                s = jnp.sum(u0 * v0 + u1 * v1)
                res = jnp.where(lanes == i, s, res)
            out_v[j, pl.ds(g * 16, 16)] = res

    pending = start(0)
    for j in range(NCH):
        cu, cv = pending
        cu.wait()
        cv.wait()
        if j + 1 < NCH:
            pending = start(j + 1)
        compute(j)

    pltpu.sync_copy(out_v, out_hbm.at[pl.ds(base, NCH)])


@jax.jit
def _cf_sc(userIdx, servIdx, user_table, item_table):
    uidx = userIdx.astype(jnp.int32)
    sidx = servIdx.astype(jnp.int32)
    uline = (uidx // 4).reshape(NW * NCH, CHUNK)
    uoff = ((uidx % 4) * DIM).reshape(NW * NCH, CHUNK)
    sline = (sidx // 4).reshape(NW * NCH, CHUNK)
    soff = ((sidx % 4) * DIM).reshape(NW * NCH, CHUNK)

    def pack(tab):
        v = tab.shape[0]
        vpad = (v + 31) // 32 * 32
        t = jnp.pad(tab, ((0, vpad - v), (0, 0)))
        return t.reshape(vpad * DIM // LINE, LINE)

    utab = pack(user_table)
    itab = pack(item_table)

    mesh = plsc.VectorSubcoreMesh(core_axis_name="c", subcore_axis_name="s")
    out = pl.kernel(
        _body,
        out_type=jax.ShapeDtypeStruct((NW * NCH, CHUNK), jnp.float32),
        mesh=mesh,
        compiler_params=pltpu.CompilerParams(needs_layout_passes=False),
        scratch_types=[
            pltpu.VMEM((NCH, CHUNK), jnp.int32),
            pltpu.VMEM((NCH, CHUNK), jnp.int32),
            pltpu.VMEM((NCH, CHUNK), jnp.int32),
            pltpu.VMEM((NCH, CHUNK), jnp.int32),
            pltpu.VMEM((2, CHUNK, LINE), jnp.float32),
            pltpu.VMEM((2, CHUNK, LINE), jnp.float32),
            pltpu.VMEM((NCH, CHUNK), jnp.float32),
            pltpu.SemaphoreType.DMA,
        ],
    )(uline, uoff, sline, soff, utab, itab)
    return out.reshape(BATCH)


def kernel(userIdx, servIdx, user_table, item_table):
    return _cf_sc(userIdx, servIdx, user_table, item_table)
